# baseline (speedup 1.0000x reference)
import functools

import jax
import jax.numpy as jnp
from jax import lax
from jax.experimental import pallas as pl
from jax.experimental.pallas import tpu as pltpu
from jax.experimental.pallas import tpu_sc as plsc

VOCAB = 1000000
D_EMBED = 32
BATCH = 16384
HIST = 50
NB = BATCH * HIST

NC = 2
NS = 16
NW = NC * NS
BPW = NB // NW
SUB = 128
KSUB = 8
CH = SUB * KSUB
NCHUNK = BPW // CH

ROWS128 = VOCAB // 4
BLK = 10000


def _colsum_body(w_ref, out_ref):
    @pl.when(pl.program_id(0) == 0)
    def _():
        out_ref[...] = jnp.zeros_like(out_ref)

    out_ref[...] += jnp.sum(w_ref[...], axis=0, keepdims=True)


def _col_sums(w128):
    return pl.pallas_call(
        _colsum_body,
        grid=(ROWS128 // BLK,),
        in_specs=[pl.BlockSpec((BLK, 128), lambda i: (i, 0))],
        out_specs=pl.BlockSpec((1, 128), lambda i: (0, 0)),
        out_shape=jax.ShapeDtypeStruct((1, 128), jnp.float32),
    )(w128)


_MESH = plsc.VectorSubcoreMesh(core_axis_name="c", subcore_axis_name="s")


@functools.partial(
    pl.kernel,
    mesh=_MESH,
    compiler_params=pltpu.CompilerParams(use_tc_tiling_on_sc=False),
    out_type=jax.ShapeDtypeStruct((NB, D_EMBED), jnp.float32),
    scratch_types=[
        pltpu.VMEM((KSUB, SUB), jnp.int32),
        pltpu.VMEM((CH, D_EMBED), jnp.float32),
        pltpu.VMEM((128,), jnp.float32),
        pltpu.SemaphoreType.DMA,
    ],
)
def _gather_sub(x_hbm, sums_hbm, table_hbm, out_hbm, idx_v, rows_v, sums_v, sem):
    wid = lax.axis_index("s") * NC + lax.axis_index("c")
    base = wid * BPW
    xrow = wid * (BPW // SUB)

    pltpu.sync_copy(sums_hbm, sums_v)
    scale = -1.0 / VOCAB
    nm0 = (sums_v[pl.ds(0, 16)] + sums_v[pl.ds(32, 16)]
           + sums_v[pl.ds(64, 16)] + sums_v[pl.ds(96, 16)]) * scale
    nm1 = (sums_v[pl.ds(16, 16)] + sums_v[pl.ds(48, 16)]
           + sums_v[pl.ds(80, 16)] + sums_v[pl.ds(112, 16)]) * scale

    def chunk(g, carry):
        off = base + g * CH
        pltpu.sync_copy(x_hbm.at[pl.ds(xrow + g * KSUB, KSUB)], idx_v)
        copies = [
            pltpu.async_copy(
                table_hbm.at[idx_v.at[j]],
                rows_v.at[pl.ds(j * SUB, SUB)],
                sem,
            )
            for j in range(KSUB)
        ]
        for c in copies:
            c.wait()

        def row(i, c):
            plsc.addupdate(rows_v.at[i, pl.ds(0, 16)], nm0)
            plsc.addupdate(rows_v.at[i, pl.ds(16, 16)], nm1)
            return c

        lax.fori_loop(0, CH, row, 0, unroll=8)
        pltpu.sync_copy(rows_v, out_hbm.at[pl.ds(off, CH)])
        return carry

    lax.fori_loop(0, NCHUNK, chunk, 0)


def kernel(x, weight):
    x2d = x.reshape(NB // SUB, SUB).astype(jnp.int32)
    w128 = weight.reshape(ROWS128, 128)
    sums = _col_sums(w128).reshape(128)
    out = _gather_sub(x2d, sums, weight)
    return out.reshape(BATCH, HIST, D_EMBED)


# speedup vs baseline: 1.7669x; 1.7669x over previous
import functools

import jax
import jax.numpy as jnp
from jax import lax
from jax.experimental import pallas as pl
from jax.experimental.pallas import tpu as pltpu
from jax.experimental.pallas import tpu_sc as plsc

VOCAB = 1000000
D_EMBED = 32
BATCH = 16384
HIST = 50
NB = BATCH * HIST

NC = 2
NS = 16
NW = NC * NS
BW = BATCH // NW
BCH = 16
NCHUNK = BW // BCH

RPT = VOCAB // NS
TCH = 1250
NTCH = RPT // TCH

_MESH = plsc.VectorSubcoreMesh(core_axis_name="c", subcore_axis_name="s")


def _acc_rows(buf, n, acc, unroll=8):
    def row(i, c):
        c0, c1 = c
        return (c0 + buf[i, pl.ds(0, 16)], c1 + buf[i, pl.ds(16, 16)])

    return lax.fori_loop(0, n, row, acc, unroll=unroll)


@functools.partial(
    pl.kernel,
    mesh=_MESH,
    compiler_params=pltpu.CompilerParams(use_tc_tiling_on_sc=False),
    out_type=jax.ShapeDtypeStruct((BATCH, HIST, D_EMBED), jnp.float32),
    scratch_types=[
        pltpu.VMEM((TCH, D_EMBED), jnp.float32),
        pltpu.VMEM((TCH, D_EMBED), jnp.float32),
        pltpu.VMEM((32,), jnp.float32),
        pltpu.VMEM((NS, 32), jnp.float32),
        pltpu.VMEM_SHARED((NS, 32), jnp.float32),
        pltpu.VMEM((BCH, HIST), jnp.int32),
        pltpu.VMEM((BCH, HIST, D_EMBED), jnp.float32),
        pltpu.SemaphoreType.DMA,
        pltpu.SemaphoreType.DMA,
    ],
)
def _zme(x_hbm, table_hbm, out3_hbm,
         tbuf0, tbuf1, part_v, all_v, shared, idx_v, rows3_v, sem1, sem2):
    sid = lax.axis_index("s")
    wid = sid * NC + lax.axis_index("c")

    row0 = sid * RPT
    pltpu.async_copy(table_hbm.at[pl.ds(row0, TCH)], tbuf0, sem1)
    zero = jnp.zeros((16,), jnp.float32)

    def two(g, acc):
        pltpu.async_copy(table_hbm.at[pl.ds(row0 + (2 * g + 1) * TCH, TCH)],
                         tbuf1, sem1)
        pltpu.make_async_copy(table_hbm.at[pl.ds(row0, TCH)], tbuf0, sem1).wait()
        acc = _acc_rows(tbuf0, TCH, acc)

        @pl.when(g < NTCH // 2 - 1)
        def _():
            pltpu.async_copy(table_hbm.at[pl.ds(row0 + (2 * g + 2) * TCH, TCH)],
                             tbuf0, sem1)

        pltpu.make_async_copy(table_hbm.at[pl.ds(row0, TCH)], tbuf1, sem1).wait()
        return _acc_rows(tbuf1, TCH, acc)

    a0, a1 = lax.fori_loop(0, NTCH // 2, two, (zero, zero))
    part_v[pl.ds(0, 16)] = a0
    part_v[pl.ds(16, 16)] = a1

    pltpu.sync_copy(part_v, shared.at[sid])
    plsc.subcore_barrier()
    pltpu.sync_copy(shared, all_v)

    def fold(i, c):
        c0, c1 = c
        return (c0 + all_v[i, pl.ds(0, 16)], c1 + all_v[i, pl.ds(16, 16)])

    scale = -1.0 / VOCAB
    m0, m1 = lax.fori_loop(0, NS, fold, (zero, zero), unroll=4)
    nm0 = m0 * scale
    nm1 = m1 * scale

    base = wid * BW

    def chunk(g, carry):
        b0 = base + g * BCH
        pltpu.sync_copy(x_hbm.at[pl.ds(b0, BCH)], idx_v)
        copies = [
            pltpu.async_copy(
                table_hbm.at[idx_v.at[j]],
                rows3_v.at[j],
                sem2,
            )
            for j in range(BCH)
        ]
        for c in copies:
            c.wait()

        def brow(j, c):
            def hrow(h, c2):
                plsc.addupdate(rows3_v.at[j, h, pl.ds(0, 16)], nm0)
                plsc.addupdate(rows3_v.at[j, h, pl.ds(16, 16)], nm1)
                return c2

            return lax.fori_loop(0, HIST, hrow, c, unroll=10)

        lax.fori_loop(0, BCH, brow, 0)
        pltpu.sync_copy(rows3_v, out3_hbm.at[pl.ds(b0, BCH)])
        return carry

    lax.fori_loop(0, NCHUNK, chunk, 0)


def kernel(x, weight):
    return _zme(x.astype(jnp.int32), weight)


# speedup vs baseline: 1.8180x; 1.0290x over previous
import functools

import jax
import jax.numpy as jnp
from jax import lax
from jax.experimental import pallas as pl
from jax.experimental.pallas import tpu as pltpu
from jax.experimental.pallas import tpu_sc as plsc

VOCAB = 1000000
D_EMBED = 32
BATCH = 16384
HIST = 50

NC = 2
NS = 16
NW = NC * NS

RPT = VOCAB // NS
TCH = 625
NTCH = RPT // TCH

BCH = 1024
NBCH = BATCH // BCH
SUB = 128
KSUB = BCH // SUB

_MESH = plsc.VectorSubcoreMesh(core_axis_name="c", subcore_axis_name="s")


def _acc_rows(buf, n, acc, unroll=8):
    def row(i, c):
        c0, c1 = c
        return (c0 + buf[i, pl.ds(0, 16)], c1 + buf[i, pl.ds(16, 16)])

    return lax.fori_loop(0, n, row, acc, unroll=unroll)


@functools.partial(
    pl.kernel,
    mesh=_MESH,
    compiler_params=pltpu.CompilerParams(
        use_tc_tiling_on_sc=False, needs_layout_passes=False),
    out_type=jax.ShapeDtypeStruct((HIST, D_EMBED, BATCH), jnp.float32),
    scratch_types=[
        pltpu.VMEM((TCH, D_EMBED), jnp.float32),
        pltpu.VMEM((TCH, D_EMBED), jnp.float32),
        pltpu.VMEM((32,), jnp.float32),
        pltpu.VMEM((NS, 32), jnp.float32),
        pltpu.VMEM_SHARED((NS, 32), jnp.float32),
        pltpu.VMEM((KSUB, SUB), jnp.int32),
        pltpu.VMEM((BCH, D_EMBED), jnp.float32),
        pltpu.VMEM((D_EMBED, BCH + 1), jnp.float32),
        pltpu.SemaphoreType.DMA,
        pltpu.SemaphoreType.DMA,
    ],
)
def _zme(x_hbm, table_hbm, out_hbm,
         tbuf0, tbuf1, part_v, all_v, shared, idx_v, rows_v, trows_v,
         sem1, sem2):
    sid = lax.axis_index("s")
    wid = sid * NC + lax.axis_index("c")

    row0 = sid * RPT
    pltpu.async_copy(table_hbm.at[pl.ds(row0, TCH)], tbuf0, sem1)
    zero = jnp.zeros((16,), jnp.float32)

    def two(g, acc):
        pltpu.async_copy(table_hbm.at[pl.ds(row0 + (2 * g + 1) * TCH, TCH)],
                         tbuf1, sem1)
        pltpu.make_async_copy(table_hbm.at[pl.ds(row0, TCH)], tbuf0, sem1).wait()
        acc = _acc_rows(tbuf0, TCH, acc)

        @pl.when(g < NTCH // 2 - 1)
        def _():
            pltpu.async_copy(table_hbm.at[pl.ds(row0 + (2 * g + 2) * TCH, TCH)],
                             tbuf0, sem1)

        pltpu.make_async_copy(table_hbm.at[pl.ds(row0, TCH)], tbuf1, sem1).wait()
        return _acc_rows(tbuf1, TCH, acc)

    a0, a1 = lax.fori_loop(0, NTCH // 2, two, (zero, zero))
    part_v[pl.ds(0, 16)] = a0
    part_v[pl.ds(16, 16)] = a1

    pltpu.sync_copy(part_v, shared.at[sid])
    plsc.subcore_barrier()
    pltpu.sync_copy(shared, all_v)

    def fold(i, c):
        c0, c1 = c
        return (c0 + all_v[i, pl.ds(0, 16)], c1 + all_v[i, pl.ds(16, 16)])

    scale = -1.0 / VOCAB
    m0, m1 = lax.fori_loop(0, NS, fold, (zero, zero), unroll=4)
    nm0 = m0 * scale
    nm1 = m1 * scale

    lanes = lax.iota(jnp.int32, 16)

    def do_h(h):
        def chunk(bg, carry):
            pltpu.sync_copy(x_hbm.at[h, pl.ds(bg * KSUB, KSUB)], idx_v)
            copies = [
                pltpu.async_copy(
                    table_hbm.at[idx_v.at[j]],
                    rows_v.at[pl.ds(j * SUB, SUB)],
                    sem2,
                )
                for j in range(KSUB)
            ]
            for c in copies:
                c.wait()

            def row(i, c):
                col = jnp.full((16,), i, jnp.int32)
                plsc.store_scatter(trows_v, [lanes, col],
                                   rows_v[i, pl.ds(0, 16)] + nm0)
                plsc.store_scatter(trows_v, [lanes + 16, col],
                                   rows_v[i, pl.ds(16, 16)] + nm1)
                return c

            lax.fori_loop(0, BCH, row, 0, unroll=8)
            pltpu.sync_copy(trows_v.at[:, pl.ds(0, BCH)],
                            out_hbm.at[h, :, pl.ds(bg * BCH, BCH)])
            return carry

        lax.fori_loop(0, NBCH, chunk, 0)

    for hi in range(2):
        h = wid + NW * hi

        @pl.when(h < HIST)
        def _():
            do_h(h)


def kernel(x, weight):
    xt3 = x.T.reshape(HIST, BATCH // SUB, SUB).astype(jnp.int32)
    out = _zme(xt3, weight)
    return jnp.transpose(out, (2, 0, 1))


# speedup vs baseline: 1.9685x; 1.0828x over previous
import functools

import jax
import jax.numpy as jnp
from jax import lax
from jax.experimental import pallas as pl
from jax.experimental.pallas import tpu as pltpu
from jax.experimental.pallas import tpu_sc as plsc

VOCAB = 1000000
D_EMBED = 32
BATCH = 16384
HIST = 50

NC = 2
NS = 16
NW = NC * NS

RPT = VOCAB // NS
TCH = 625
NTCH = RPT // TCH

BCH = 1024
NBCH = BATCH // BCH
SUB = 128
KSUB = BCH // SUB

_MESH = plsc.VectorSubcoreMesh(core_axis_name="c", subcore_axis_name="s")


def _acc_rows(buf, n, acc, unroll=8):
    def row(i, c):
        c0, c1 = c
        return (c0 + buf[i, pl.ds(0, 16)], c1 + buf[i, pl.ds(16, 16)])

    return lax.fori_loop(0, n, row, acc, unroll=unroll)


@functools.partial(
    pl.kernel,
    mesh=_MESH,
    compiler_params=pltpu.CompilerParams(
        use_tc_tiling_on_sc=False, needs_layout_passes=False),
    out_type=jax.ShapeDtypeStruct((HIST, D_EMBED, BATCH), jnp.float32),
    scratch_types=[
        pltpu.VMEM((TCH, D_EMBED), jnp.float32),
        pltpu.VMEM((TCH, D_EMBED), jnp.float32),
        pltpu.VMEM((32,), jnp.float32),
        pltpu.VMEM((NS, 32), jnp.float32),
        pltpu.VMEM_SHARED((NS, 32), jnp.float32),
        pltpu.VMEM((KSUB, SUB), jnp.int32),
        pltpu.VMEM((BCH, D_EMBED), jnp.float32),
        pltpu.VMEM((D_EMBED, BCH + 1), jnp.float32),
        pltpu.SemaphoreType.DMA,
        pltpu.SemaphoreType.DMA,
    ],
)
def _zme(x_hbm, table_hbm, out_hbm,
         tbuf0, tbuf1, part_v, all_v, shared, idx_v, rows_v, trows_v,
         sem1, sem2):
    sid = lax.axis_index("s")
    wid = sid * NC + lax.axis_index("c")

    row0 = sid * RPT
    pltpu.async_copy(table_hbm.at[pl.ds(row0, TCH)], tbuf0, sem1)
    zero = jnp.zeros((16,), jnp.float32)

    def two(g, acc):
        pltpu.async_copy(table_hbm.at[pl.ds(row0 + (2 * g + 1) * TCH, TCH)],
                         tbuf1, sem1)
        pltpu.make_async_copy(table_hbm.at[pl.ds(row0, TCH)], tbuf0, sem1).wait()
        acc = _acc_rows(tbuf0, TCH, acc)

        @pl.when(g < NTCH // 2 - 1)
        def _():
            pltpu.async_copy(table_hbm.at[pl.ds(row0 + (2 * g + 2) * TCH, TCH)],
                             tbuf0, sem1)

        pltpu.make_async_copy(table_hbm.at[pl.ds(row0, TCH)], tbuf1, sem1).wait()
        return _acc_rows(tbuf1, TCH, acc)

    a0, a1 = lax.fori_loop(0, NTCH // 2, two, (zero, zero))
    part_v[pl.ds(0, 16)] = a0
    part_v[pl.ds(16, 16)] = a1

    pltpu.sync_copy(part_v, shared.at[sid])
    plsc.subcore_barrier()
    pltpu.sync_copy(shared, all_v)

    def fold(i, c):
        c0, c1 = c
        return (c0 + all_v[i, pl.ds(0, 16)], c1 + all_v[i, pl.ds(16, 16)])

    scale = -1.0 / VOCAB
    m0, m1 = lax.fori_loop(0, NS, fold, (zero, zero), unroll=4)
    nm0 = m0 * scale
    nm1 = m1 * scale

    lanes = lax.iota(jnp.int32, 16)

    def do_chunk(h, bg):
        pltpu.sync_copy(x_hbm.at[h, pl.ds(bg * KSUB, KSUB)], idx_v)
        copies = [
            pltpu.async_copy(
                table_hbm.at[idx_v.at[j]],
                rows_v.at[pl.ds(j * SUB, SUB)],
                sem2,
            )
            for j in range(KSUB)
        ]
        for c in copies:
            c.wait()

        def row(i, col):
            plsc.store_scatter(trows_v, [lanes, col],
                               rows_v[i, pl.ds(0, 16)] + nm0)
            plsc.store_scatter(trows_v, [lanes + 16, col],
                               rows_v[i, pl.ds(16, 16)] + nm1)
            return col + 1

        lax.fori_loop(0, BCH, row, jnp.zeros((16,), jnp.int32), unroll=8)
        pltpu.sync_copy(trows_v.at[:, pl.ds(0, BCH)],
                        out_hbm.at[h, :, pl.ds(bg * BCH, BCH)])

    def chunk1(bg, carry):
        do_chunk(wid, bg)
        return carry

    lax.fori_loop(0, NBCH, chunk1, 0)

    NCH2 = (HIST - NW) * NBCH // NW

    def chunk2(k, carry):
        cid = wid * NCH2 + k
        do_chunk(NW + cid // NBCH, cid % NBCH)
        return carry

    lax.fori_loop(0, NCH2, chunk2, 0)


def kernel(x, weight):
    xt3 = x.T.reshape(HIST, BATCH // SUB, SUB).astype(jnp.int32)
    out = _zme(xt3, weight)
    return jnp.transpose(out, (2, 0, 1))


# speedup vs baseline: 2.0340x; 1.0333x over previous
import functools

import jax
import jax.numpy as jnp
from jax import lax
from jax.experimental import pallas as pl
from jax.experimental.pallas import tpu as pltpu
from jax.experimental.pallas import tpu_sc as plsc

VOCAB = 1000000
D_EMBED = 32
BATCH = 16384
HIST = 50

NC = 2
NS = 16
NW = NC * NS

RPT = VOCAB // NS
TCH = 625
NTCH = RPT // TCH

BCH = 512
NBCH = BATCH // BCH
SUB = 128
KSUB = BCH // SUB
NCHT = HIST * NBCH // NW
NCH2 = (HIST - NW) * NBCH // NW

_MESH = plsc.VectorSubcoreMesh(core_axis_name="c", subcore_axis_name="s")


def _acc_rows(buf, n, acc, unroll=8):
    def row(i, c):
        c0, c1 = c
        return (c0 + buf[i, pl.ds(0, 16)], c1 + buf[i, pl.ds(16, 16)])

    return lax.fori_loop(0, n, row, acc, unroll=unroll)


@functools.partial(
    pl.kernel,
    mesh=_MESH,
    compiler_params=pltpu.CompilerParams(
        use_tc_tiling_on_sc=False, needs_layout_passes=False),
    out_type=jax.ShapeDtypeStruct((HIST, D_EMBED, BATCH), jnp.float32),
    scratch_types=[
        pltpu.VMEM((TCH, D_EMBED), jnp.float32),
        pltpu.VMEM((TCH, D_EMBED), jnp.float32),
        pltpu.VMEM((32,), jnp.float32),
        pltpu.VMEM((NS, 32), jnp.float32),
        pltpu.VMEM_SHARED((NS, 32), jnp.float32),
        pltpu.VMEM((KSUB, SUB), jnp.int32),
        pltpu.VMEM((KSUB, SUB), jnp.int32),
        pltpu.VMEM((BCH, D_EMBED), jnp.float32),
        pltpu.VMEM((BCH, D_EMBED), jnp.float32),
        pltpu.VMEM((D_EMBED, BCH + 1), jnp.float32),
        pltpu.SemaphoreType.DMA,
        pltpu.SemaphoreType.DMA,
    ],
)
def _zme(x_hbm, table_hbm, out_hbm,
         tbuf0, tbuf1, part_v, all_v, shared, idx_a, idx_b, rows_a, rows_b,
         trows_v, sem1, sem2):
    sid = lax.axis_index("s")
    wid = sid * NC + lax.axis_index("c")

    row0 = sid * RPT
    pltpu.async_copy(table_hbm.at[pl.ds(row0, TCH)], tbuf0, sem1)
    zero = jnp.zeros((16,), jnp.float32)

    def two(g, acc):
        pltpu.async_copy(table_hbm.at[pl.ds(row0 + (2 * g + 1) * TCH, TCH)],
                         tbuf1, sem1)
        pltpu.make_async_copy(table_hbm.at[pl.ds(row0, TCH)], tbuf0, sem1).wait()
        acc = _acc_rows(tbuf0, TCH, acc)

        @pl.when(g < NTCH // 2 - 1)
        def _():
            pltpu.async_copy(table_hbm.at[pl.ds(row0 + (2 * g + 2) * TCH, TCH)],
                             tbuf0, sem1)

        pltpu.make_async_copy(table_hbm.at[pl.ds(row0, TCH)], tbuf1, sem1).wait()
        return _acc_rows(tbuf1, TCH, acc)

    a0, a1 = lax.fori_loop(0, NTCH // 2, two, (zero, zero))
    part_v[pl.ds(0, 16)] = a0
    part_v[pl.ds(16, 16)] = a1

    pltpu.sync_copy(part_v, shared.at[sid])
    plsc.subcore_barrier()
    pltpu.sync_copy(shared, all_v)

    def fold(i, c):
        c0, c1 = c
        return (c0 + all_v[i, pl.ds(0, 16)], c1 + all_v[i, pl.ds(16, 16)])

    scale = -1.0 / VOCAB
    m0, m1 = lax.fori_loop(0, NS, fold, (zero, zero), unroll=4)
    nm0 = m0 * scale
    nm1 = m1 * scale

    lanes = lax.iota(jnp.int32, 16)

    def locate(k):
        cid = wid * NCH2 + (k - NBCH)
        h = jnp.where(k < NBCH, wid, NW + cid // NBCH)
        bg = jnp.where(k < NBCH, k, cid % NBCH)
        return h, bg

    def fire(k, idx_v, rows_v):
        h, bg = locate(k)
        pltpu.sync_copy(x_hbm.at[h, pl.ds(bg * KSUB, KSUB)], idx_v)
        for j in range(KSUB):
            pltpu.async_copy(
                table_hbm.at[idx_v.at[j]],
                rows_v.at[pl.ds(j * SUB, SUB)],
                sem2,
            )

    def process(k, rows_v):
        for j in range(KSUB):
            pltpu.make_async_copy(
                table_hbm.at[idx_a.at[j]],
                rows_v.at[pl.ds(j * SUB, SUB)],
                sem2,
            ).wait()

        def row(i, col):
            plsc.store_scatter(trows_v, [lanes, col],
                               rows_v[i, pl.ds(0, 16)] + nm0)
            plsc.store_scatter(trows_v, [lanes + 16, col],
                               rows_v[i, pl.ds(16, 16)] + nm1)
            return col + 1

        lax.fori_loop(0, BCH, row, jnp.zeros((16,), jnp.int32), unroll=8)
        h, bg = locate(k)
        pltpu.sync_copy(trows_v.at[:, pl.ds(0, BCH)],
                        out_hbm.at[h, :, pl.ds(bg * BCH, BCH)])

    fire(0, idx_a, rows_a)

    def pipelined(p, carry):
        k0 = 2 * p
        fire(k0 + 1, idx_b, rows_b)
        process(k0, rows_a)

        @pl.when(p < NCHT // 2 - 1)
        def _():
            fire(k0 + 2, idx_a, rows_a)

        process(k0 + 1, rows_b)
        return carry

    lax.fori_loop(0, NCHT // 2, pipelined, 0)


def kernel(x, weight):
    xt3 = x.T.reshape(HIST, BATCH // SUB, SUB).astype(jnp.int32)
    out = _zme(xt3, weight)
    return jnp.transpose(out, (2, 0, 1))
